# baseline (speedup 1.0000x reference)
import functools

import jax
import jax.numpy as jnp
from jax import lax
from jax.experimental import pallas as pl
from jax.experimental.pallas import tpu as pltpu
from jax.experimental.pallas import tpu_sc as plsc

N_NODES = 10000
N_EDGES = 320000
D_FEAT = 128

NC = 2
NS = 16
LANES = 16
NW = NC * NS

COLS = 128
NTILES = N_EDGES // COLS
TPW = -(-NTILES // NW)
EPW = TPW * COLS
VECS = EPW // LANES

ROW_BLK = 1024
N_BLKS = -(-N_NODES // ROW_BLK)


def _scores_tc_kernel(b_ref, x_ref, w_ref, s1_ref, s2_ref):
    res = lax.dot_general(
        w_ref[...], x_ref[...],
        dimension_numbers=(((1,), (1,)), ((), ())),
        preferred_element_type=jnp.float32,
    )
    s1_ref[...] = res[0, :] + b_ref[0, 0]
    s2_ref[...] = res[1, :]


def _node_scores(x, W, b):
    w2 = W.reshape(2, D_FEAT)
    out1d = jax.ShapeDtypeStruct((N_NODES,), jnp.float32)
    return pl.pallas_call(
        _scores_tc_kernel,
        grid=(N_BLKS,),
        in_specs=[
            pl.BlockSpec(memory_space=pltpu.SMEM),
            pl.BlockSpec((ROW_BLK, D_FEAT), lambda i: (i, 0)),
            pl.BlockSpec((2, D_FEAT), lambda i: (0, 0)),
        ],
        out_specs=[
            pl.BlockSpec((ROW_BLK,), lambda i: (i,)),
            pl.BlockSpec((ROW_BLK,), lambda i: (i,)),
        ],
        out_shape=[out1d, out1d],
    )(b.reshape(1, 1), x, w2)


def _edge_score_body(s1_hbm, s2_hbm, ei_hbm, out_hbm, s1_v, s2_v, ei_v, out_v,
                     sem):
    wid = lax.axis_index("s") * NC + lax.axis_index("c")
    t0 = jnp.minimum(wid * TPW, NTILES - TPW)
    base = t0 * COLS
    cp1 = pltpu.async_copy(s1_hbm, s1_v, sem)
    cp2 = pltpu.async_copy(s2_hbm, s2_v, sem)
    cp3 = pltpu.async_copy(ei_hbm.at[:, pl.ds(base, EPW)], ei_v, sem)
    cp1.wait()
    cp2.wait()
    cp3.wait()

    @plsc.parallel_loop(0, VECS, unroll=2)
    def _(i):
        off = pl.multiple_of(i * LANES, LANES)
        si = ei_v[0, pl.ds(off, LANES)]
        di = ei_v[1, pl.ds(off, LANES)]
        a = plsc.load_gather(s1_v, [si])
        c = plsc.load_gather(s2_v, [di])
        z = a + c
        out_v[pl.ds(off, LANES)] = 1.0 / (1.0 + jnp.exp(-z))

    pltpu.sync_copy(out_v, out_hbm.at[pl.ds(base, EPW)])


_edge_scores = functools.partial(
    pl.kernel,
    out_type=jax.ShapeDtypeStruct((N_EDGES,), jnp.float32),
    mesh=plsc.VectorSubcoreMesh(
        core_axis_name="c", subcore_axis_name="s", num_cores=NC,
        num_subcores=NS,
    ),
    scratch_types=[
        pltpu.VMEM((N_NODES,), jnp.float32),
        pltpu.VMEM((N_NODES,), jnp.float32),
        pltpu.VMEM((2, EPW), jnp.int32),
        pltpu.VMEM((EPW,), jnp.float32),
        pltpu.SemaphoreType.DMA,
    ],
    compiler_params=pltpu.CompilerParams(needs_layout_passes=False, skip_device_barrier=True),
)(_edge_score_body)


def kernel(x, edge_index, W, b):
    s1, s2 = _node_scores(x, W, b)
    return _edge_scores(s1, s2, edge_index)


# speedup vs baseline: 1.1103x; 1.1103x over previous
import functools

import jax
import jax.numpy as jnp
from jax import lax
from jax.experimental import pallas as pl
from jax.experimental.pallas import tpu as pltpu
from jax.experimental.pallas import tpu_sc as plsc

N_NODES = 10000
N_EDGES = 320000
D_FEAT = 128

NC = 2
NS = 16
LANES = 16
NW = NC * NS

COLS = 128
NTILES = N_EDGES // COLS
TPW = -(-NTILES // NW)
EPW = TPW * COLS
VECS = EPW // LANES

ROW_BLK = 4096
N_BLKS = -(-N_NODES // ROW_BLK)


def _scores_tc_kernel(b_ref, x_ref, w_ref, s1_ref, s2_ref):
    res = lax.dot_general(
        w_ref[...], x_ref[...],
        dimension_numbers=(((1,), (1,)), ((), ())),
        preferred_element_type=jnp.float32,
    )
    s1_ref[...] = res[0, :] + b_ref[0, 0]
    s2_ref[...] = res[1, :]


def _node_scores(x, W, b):
    w2 = W.reshape(2, D_FEAT)
    out1d = jax.ShapeDtypeStruct((N_NODES,), jnp.float32)
    return pl.pallas_call(
        _scores_tc_kernel,
        grid=(N_BLKS,),
        in_specs=[
            pl.BlockSpec(memory_space=pltpu.SMEM),
            pl.BlockSpec((ROW_BLK, D_FEAT), lambda i: (i, 0)),
            pl.BlockSpec((2, D_FEAT), lambda i: (0, 0)),
        ],
        out_specs=[
            pl.BlockSpec((ROW_BLK,), lambda i: (i,)),
            pl.BlockSpec((ROW_BLK,), lambda i: (i,)),
        ],
        out_shape=[out1d, out1d],
    )(b.reshape(1, 1), x, w2)


def _edge_score_body(s1_hbm, s2_hbm, ei_hbm, out_hbm, s1_v, s2_v, ei_v, out_v,
                     sem):
    wid = lax.axis_index("s") * NC + lax.axis_index("c")
    t0 = jnp.minimum(wid * TPW, NTILES - TPW)
    base = t0 * COLS
    cp1 = pltpu.async_copy(s1_hbm, s1_v, sem)
    cp2 = pltpu.async_copy(s2_hbm, s2_v, sem)
    cp3 = pltpu.async_copy(ei_hbm.at[:, pl.ds(base, EPW)], ei_v, sem)
    cp1.wait()
    cp2.wait()
    cp3.wait()

    @plsc.parallel_loop(0, VECS, unroll=2)
    def _(i):
        off = pl.multiple_of(i * LANES, LANES)
        si = ei_v[0, pl.ds(off, LANES)]
        di = ei_v[1, pl.ds(off, LANES)]
        a = plsc.load_gather(s1_v, [si])
        c = plsc.load_gather(s2_v, [di])
        z = a + c
        out_v[pl.ds(off, LANES)] = 1.0 / (1.0 + jnp.exp(-z))

    pltpu.sync_copy(out_v, out_hbm.at[pl.ds(base, EPW)])


_edge_scores = functools.partial(
    pl.kernel,
    out_type=jax.ShapeDtypeStruct((N_EDGES,), jnp.float32),
    mesh=plsc.VectorSubcoreMesh(
        core_axis_name="c", subcore_axis_name="s", num_cores=NC,
        num_subcores=NS,
    ),
    scratch_types=[
        pltpu.VMEM((N_NODES,), jnp.float32),
        pltpu.VMEM((N_NODES,), jnp.float32),
        pltpu.VMEM((2, EPW), jnp.int32),
        pltpu.VMEM((EPW,), jnp.float32),
        pltpu.SemaphoreType.DMA,
    ],
    compiler_params=pltpu.CompilerParams(needs_layout_passes=False),
)(_edge_score_body)


def kernel(x, edge_index, W, b):
    s1, s2 = _node_scores(x, W, b)
    return _edge_scores(s1, s2, edge_index)


# speedup vs baseline: 1.1161x; 1.0052x over previous
import functools

import jax
import jax.numpy as jnp
from jax import lax
from jax.experimental import pallas as pl
from jax.experimental.pallas import tpu as pltpu
from jax.experimental.pallas import tpu_sc as plsc

N_NODES = 10000
N_EDGES = 320000
D_FEAT = 128

NC = 2
NS = 16
LANES = 16
NW = NC * NS

COLS = 128
NTILES = N_EDGES // COLS
TPW = -(-NTILES // NW)
EPW = TPW * COLS
VECS = EPW // LANES

ROW_BLK = 8192
N_BLKS = -(-N_NODES // ROW_BLK)


def _scores_tc_kernel(b_ref, x_ref, w_ref, s1_ref, s2_ref):
    res = lax.dot_general(
        w_ref[...], x_ref[...],
        dimension_numbers=(((1,), (1,)), ((), ())),
        preferred_element_type=jnp.float32,
    )
    s1_ref[...] = res[0, :] + b_ref[0, 0]
    s2_ref[...] = res[1, :]


def _node_scores(x, W, b):
    w2 = W.reshape(2, D_FEAT)
    out1d = jax.ShapeDtypeStruct((N_NODES,), jnp.float32)
    return pl.pallas_call(
        _scores_tc_kernel,
        grid=(N_BLKS,),
        in_specs=[
            pl.BlockSpec(memory_space=pltpu.SMEM),
            pl.BlockSpec((ROW_BLK, D_FEAT), lambda i: (i, 0)),
            pl.BlockSpec((2, D_FEAT), lambda i: (0, 0)),
        ],
        out_specs=[
            pl.BlockSpec((ROW_BLK,), lambda i: (i,)),
            pl.BlockSpec((ROW_BLK,), lambda i: (i,)),
        ],
        out_shape=[out1d, out1d],
    )(b.reshape(1, 1), x, w2)


def _edge_score_body(s1_hbm, s2_hbm, ei_hbm, out_hbm, s1_v, s2_v, ei_v, out_v,
                     sem):
    wid = lax.axis_index("s") * NC + lax.axis_index("c")
    t0 = jnp.minimum(wid * TPW, NTILES - TPW)
    base = t0 * COLS
    cp1 = pltpu.async_copy(s1_hbm, s1_v, sem)
    cp2 = pltpu.async_copy(s2_hbm, s2_v, sem)
    cp3 = pltpu.async_copy(ei_hbm.at[:, pl.ds(base, EPW)], ei_v, sem)
    cp1.wait()
    cp2.wait()
    cp3.wait()

    @plsc.parallel_loop(0, VECS, unroll=2)
    def _(i):
        off = pl.multiple_of(i * LANES, LANES)
        si = ei_v[0, pl.ds(off, LANES)]
        di = ei_v[1, pl.ds(off, LANES)]
        a = plsc.load_gather(s1_v, [si])
        c = plsc.load_gather(s2_v, [di])
        z = a + c
        out_v[pl.ds(off, LANES)] = 1.0 / (1.0 + jnp.exp(-z))

    pltpu.sync_copy(out_v, out_hbm.at[pl.ds(base, EPW)])


_edge_scores = functools.partial(
    pl.kernel,
    out_type=jax.ShapeDtypeStruct((N_EDGES,), jnp.float32),
    mesh=plsc.VectorSubcoreMesh(
        core_axis_name="c", subcore_axis_name="s", num_cores=NC,
        num_subcores=NS,
    ),
    scratch_types=[
        pltpu.VMEM((N_NODES,), jnp.float32),
        pltpu.VMEM((N_NODES,), jnp.float32),
        pltpu.VMEM((2, EPW), jnp.int32),
        pltpu.VMEM((EPW,), jnp.float32),
        pltpu.SemaphoreType.DMA,
    ],
    compiler_params=pltpu.CompilerParams(needs_layout_passes=False),
)(_edge_score_body)


def kernel(x, edge_index, W, b):
    s1, s2 = _node_scores(x, W, b)
    return _edge_scores(s1, s2, edge_index)
